# baseline (device time: 24867 ns/iter reference)
import os

import jax
import jax.numpy as jnp
from jax import lax
from jax.experimental import pallas as pl
from jax.experimental.pallas import tpu as pltpu

N_CHUNKS = 16

_DIAG = int(os.environ.get("KERNEL_DIAG", "0"))


def kernel(x):
    m, n = x.shape
    half = m // 2
    chunk = half // N_CHUNKS

    def body(x_ref, out_ref, stage_ref, send_ref, recv_ref, fetch_sems,
             send_sems1, recv_sems1, send_sems2, recv_sems2):
        my_x = lax.axis_index("x")
        my_y = lax.axis_index("y")
        x_nbr = (1 - my_x, my_y)
        y_nbr = (my_x, 1 - my_y)
        row0 = my_y * half

        fetches = []
        for k in range(N_CHUNKS):
            f = pltpu.make_async_copy(
                x_ref.at[pl.ds(row0 + k * chunk, chunk), :],
                stage_ref.at[pl.ds(k * chunk, chunk), :],
                fetch_sems.at[k],
            )
            f.start()
            fetches.append(f)

        if _DIAG != 2:
            nbrs = (x_nbr,) if _DIAG == 1 else (x_nbr, y_nbr)
            barrier_sem = pltpu.get_barrier_semaphore()
            for nbr in nbrs:
                pl.semaphore_signal(
                    barrier_sem, inc=1,
                    device_id=nbr, device_id_type=pl.DeviceIdType.MESH,
                )
            pl.semaphore_wait(barrier_sem, len(nbrs))

        rdma1 = []
        for k in range(N_CHUNKS):
            rows_loc = pl.ds(k * chunk, chunk)
            fetches[k].wait()
            send_ref[rows_loc, :] = stage_ref[rows_loc, :].astype(jnp.bfloat16)
            if _DIAG == 2:
                continue
            r = pltpu.make_async_remote_copy(
                src_ref=send_ref.at[rows_loc, :],
                dst_ref=recv_ref.at[rows_loc, :],
                send_sem=send_sems1.at[k],
                recv_sem=recv_sems1.at[k],
                device_id=x_nbr,
                device_id_type=pl.DeviceIdType.MESH,
            )
            r.start()
            rdma1.append(r)

        if _DIAG == 2:
            for k in range(2 * N_CHUNKS):
                rows = pl.ds(k * chunk, chunk)
                out_ref[rows, :] = (
                    send_ref[pl.ds((k % N_CHUNKS) * chunk, chunk), :] * 2.0
                )
            return

        rdma2 = []
        for k in range(N_CHUNKS):
            rdma1[k].wait_recv()
            rows_loc = pl.ds(k * chunk, chunk)
            rows = pl.ds(row0 + k * chunk, chunk)
            out_ref[rows, :] = send_ref[rows_loc, :] + recv_ref[rows_loc, :]
            if _DIAG == 0:
                r = pltpu.make_async_remote_copy(
                    src_ref=out_ref.at[rows, :],
                    dst_ref=out_ref.at[rows, :],
                    send_sem=send_sems2.at[k],
                    recv_sem=recv_sems2.at[k],
                    device_id=y_nbr,
                    device_id_type=pl.DeviceIdType.MESH,
                )
                r.start()
                rdma2.append(r)
            else:
                other = pl.ds((half - row0) + k * chunk, chunk)
                out_ref[other, :] = (
                    send_ref[rows_loc, :] + recv_ref[rows_loc, :]
                )

        for k in range(N_CHUNKS):
            rdma1[k].wait_send()
            if _DIAG == 0:
                rdma2[k].wait()

    return pl.pallas_call(
        body,
        out_shape=jax.ShapeDtypeStruct((m, n), jnp.bfloat16),
        in_specs=[pl.BlockSpec(memory_space=pl.ANY)],
        out_specs=pl.BlockSpec(memory_space=pltpu.VMEM),
        scratch_shapes=[
            pltpu.VMEM((half, n), jnp.float32),
            pltpu.VMEM((half, n), jnp.bfloat16),
            pltpu.VMEM((half, n), jnp.bfloat16),
            pltpu.SemaphoreType.DMA((N_CHUNKS,)),
            pltpu.SemaphoreType.DMA((N_CHUNKS,)),
            pltpu.SemaphoreType.DMA((N_CHUNKS,)),
            pltpu.SemaphoreType.DMA((N_CHUNKS,)),
            pltpu.SemaphoreType.DMA((N_CHUNKS,)),
        ],
        compiler_params=pltpu.CompilerParams(
            collective_id=None if _DIAG == 2 else 0
        ),
    )(x)


# device time: 22477 ns/iter; 1.1063x vs baseline; 1.1063x over previous
import jax
import jax.numpy as jnp
from jax import lax
from jax.experimental import pallas as pl
from jax.experimental.pallas import tpu as pltpu

N_CHUNKS = 16


def kernel(x):
    m, n = x.shape
    half = m // 2
    chunk = half // N_CHUNKS

    def body(x_ref, out_ref, stage_ref, send_ref, recv_ref, red_ref,
             fetch_sem, store_sems,
             send_sems1, recv_sems1, send_sems2, recv_sems2):
        my_x = lax.axis_index("x")
        my_y = lax.axis_index("y")
        x_nbr = (1 - my_x, my_y)
        y_nbr = (my_x, 1 - my_y)
        row0 = my_y * half

        fetch = pltpu.make_async_copy(
            x_ref.at[pl.ds(row0, half), :], stage_ref, fetch_sem
        )
        fetch.start()

        barrier_sem = pltpu.get_barrier_semaphore()
        for nbr in (x_nbr, y_nbr):
            pl.semaphore_signal(
                barrier_sem, inc=1,
                device_id=nbr, device_id_type=pl.DeviceIdType.MESH,
            )
        pl.semaphore_wait(barrier_sem, 2)

        fetch.wait()
        send_ref[:, :] = stage_ref[:, :].astype(jnp.bfloat16)

        rdma1 = []
        for k in range(N_CHUNKS):
            rows_loc = pl.ds(k * chunk, chunk)
            r = pltpu.make_async_remote_copy(
                src_ref=send_ref.at[rows_loc, :],
                dst_ref=recv_ref.at[rows_loc, :],
                send_sem=send_sems1.at[k],
                recv_sem=recv_sems1.at[k],
                device_id=x_nbr,
                device_id_type=pl.DeviceIdType.MESH,
            )
            r.start()
            rdma1.append(r)

        rdma2 = []
        stores = []
        for k in range(N_CHUNKS):
            rdma1[k].wait_recv()
            rows_loc = pl.ds(k * chunk, chunk)
            rows = pl.ds(row0 + k * chunk, chunk)
            red_ref[rows_loc, :] = send_ref[rows_loc, :] + recv_ref[rows_loc, :]
            r = pltpu.make_async_remote_copy(
                src_ref=red_ref.at[rows_loc, :],
                dst_ref=out_ref.at[rows, :],
                send_sem=send_sems2.at[k],
                recv_sem=recv_sems2.at[k],
                device_id=y_nbr,
                device_id_type=pl.DeviceIdType.MESH,
            )
            r.start()
            rdma2.append(r)
            s = pltpu.make_async_copy(
                red_ref.at[rows_loc, :], out_ref.at[rows, :], store_sems.at[k]
            )
            s.start()
            stores.append(s)

        for k in range(N_CHUNKS):
            rdma1[k].wait_send()
            stores[k].wait()
            rdma2[k].wait()

    return pl.pallas_call(
        body,
        out_shape=jax.ShapeDtypeStruct((m, n), jnp.bfloat16),
        in_specs=[pl.BlockSpec(memory_space=pltpu.MemorySpace.HBM)],
        out_specs=pl.BlockSpec(memory_space=pltpu.MemorySpace.HBM),
        scratch_shapes=[
            pltpu.VMEM((half, n), jnp.float32),
            pltpu.VMEM((half, n), jnp.bfloat16),
            pltpu.VMEM((half, n), jnp.bfloat16),
            pltpu.VMEM((half, n), jnp.bfloat16),
            pltpu.SemaphoreType.DMA,
            pltpu.SemaphoreType.DMA((N_CHUNKS,)),
            pltpu.SemaphoreType.DMA((N_CHUNKS,)),
            pltpu.SemaphoreType.DMA((N_CHUNKS,)),
            pltpu.SemaphoreType.DMA((N_CHUNKS,)),
            pltpu.SemaphoreType.DMA((N_CHUNKS,)),
        ],
        compiler_params=pltpu.CompilerParams(collective_id=0),
    )(x)


# device time: 20181 ns/iter; 1.2322x vs baseline; 1.1138x over previous
import jax
import jax.numpy as jnp
from jax import lax
from jax.experimental import pallas as pl
from jax.experimental.pallas import tpu as pltpu

N_CHUNKS = 16


def kernel(x):
    m, n = x.shape
    half = m // 2
    chunk = half // N_CHUNKS

    my_y_out = lax.axis_index("y")
    x_half = lax.dynamic_slice_in_dim(x, my_y_out * half, half, axis=0)

    def body(x_ref, out_ref, send_ref, recv_ref,
             send_sems1, recv_sems1, send_sems2, recv_sems2):
        my_x = lax.axis_index("x")
        my_y = lax.axis_index("y")
        x_nbr = (1 - my_x, my_y)
        y_nbr = (my_x, 1 - my_y)
        row0 = my_y * half

        send_ref[:, :] = x_ref[:, :].astype(jnp.bfloat16)

        barrier_sem = pltpu.get_barrier_semaphore()
        for nbr in (x_nbr, y_nbr):
            pl.semaphore_signal(
                barrier_sem, inc=1,
                device_id=nbr, device_id_type=pl.DeviceIdType.MESH,
            )
        pl.semaphore_wait(barrier_sem, 2)

        rdma1 = []
        for k in range(N_CHUNKS):
            rows_loc = pl.ds(k * chunk, chunk)
            r = pltpu.make_async_remote_copy(
                src_ref=send_ref.at[rows_loc, :],
                dst_ref=recv_ref.at[rows_loc, :],
                send_sem=send_sems1.at[k],
                recv_sem=recv_sems1.at[k],
                device_id=x_nbr,
                device_id_type=pl.DeviceIdType.MESH,
            )
            r.start()
            rdma1.append(r)

        rdma2 = []
        for k in range(N_CHUNKS):
            rdma1[k].wait_recv()
            rows_loc = pl.ds(k * chunk, chunk)
            rows = pl.ds(row0 + k * chunk, chunk)
            out_ref[rows, :] = send_ref[rows_loc, :] + recv_ref[rows_loc, :]
            r = pltpu.make_async_remote_copy(
                src_ref=out_ref.at[rows, :],
                dst_ref=out_ref.at[rows, :],
                send_sem=send_sems2.at[k],
                recv_sem=recv_sems2.at[k],
                device_id=y_nbr,
                device_id_type=pl.DeviceIdType.MESH,
            )
            r.start()
            rdma2.append(r)

        for k in range(N_CHUNKS):
            rdma1[k].wait_send()
            rdma2[k].wait()

    return pl.pallas_call(
        body,
        out_shape=jax.ShapeDtypeStruct((m, n), jnp.bfloat16),
        in_specs=[pl.BlockSpec(memory_space=pltpu.VMEM)],
        out_specs=pl.BlockSpec(memory_space=pltpu.VMEM),
        scratch_shapes=[
            pltpu.VMEM((half, n), jnp.bfloat16),
            pltpu.VMEM((half, n), jnp.bfloat16),
            pltpu.SemaphoreType.DMA((N_CHUNKS,)),
            pltpu.SemaphoreType.DMA((N_CHUNKS,)),
            pltpu.SemaphoreType.DMA((N_CHUNKS,)),
            pltpu.SemaphoreType.DMA((N_CHUNKS,)),
        ],
        compiler_params=pltpu.CompilerParams(collective_id=0),
    )(x_half)
